# Initial kernel scaffold; baseline (speedup 1.0000x reference)
import functools
import jax
import jax.numpy as jnp
from jax import lax
from jax.experimental import pallas as pl
from jax.experimental.pallas import tpu as pltpu
from jax.experimental.pallas import tpu_sc as plsc

L = 16


def _rsqrt16(x):
    i = plsc.bitcast(x, jnp.int32)
    i = jnp.full((L,), 0x5F3759DF, dtype=jnp.int32) - lax.shift_right_logical(
        i, jnp.full((L,), 1, dtype=jnp.int32))
    y = plsc.bitcast(i, jnp.float32)
    half = jnp.full((L,), 0.5, dtype=jnp.float32)
    three_half = jnp.full((L,), 1.5, dtype=jnp.float32)
    hx = half * x
    for _ in range(3):
        y = y * (three_half - hx * y * y)
    return y


def _make_kernel(B, S, H, V, eps):
    assert H == 4 * L
    info = plsc.get_sparse_core_info()
    nw = info.num_cores * info.num_subcores
    assert B % nw == 0
    rows_per_w = B // nw
    c_len = ((S + 1) // 2 + 7) // 8 * 8
    s2 = S - c_len
    assert s2 >= 0 and s2 % 8 == 0 and c_len <= 128 and S % 8 == 0

    mesh = plsc.VectorSubcoreMesh(core_axis_name="c", subcore_axis_name="s")

    @functools.partial(
        pl.kernel,
        out_type=jax.ShapeDtypeStruct((B * S * H,), jnp.float32),
        mesh=mesh,
        scratch_types=[
            pltpu.VMEM((2, c_len), jnp.int32),
            pltpu.VMEM((S,), jnp.int32),
            pltpu.VMEM((S, H), jnp.float32),
            pltpu.VMEM((S * H,), jnp.float32),
            pltpu.VMEM((2 * H,), jnp.float32),
            pltpu.VMEM((H,), jnp.float32),
            pltpu.VMEM((H,), jnp.float32),
            pltpu.VMEM((S * H,), jnp.float32),
            pltpu.SemaphoreType.DMA,
        ],
    )
    def k(ids_hbm, tt_hbm, tok_table_hbm, pos_hbm, typ_hbm, g_hbm, b_hbm,
          out_hbm, idx_v, tt_v, tok_v, pos_v, typ_v, g_v, b_v, out_v, sem):
        wid = lax.axis_index("s") * info.num_cores + lax.axis_index("c")
        row0 = wid * rows_per_w

        pltpu.sync_copy(pos_hbm, pos_v)
        pltpu.sync_copy(typ_hbm, typ_v)
        pltpu.sync_copy(g_hbm, g_v)
        pltpu.sync_copy(b_hbm, b_v)

        t0 = [typ_v[pl.ds(g * L, L)] for g in range(4)]
        td = [typ_v[pl.ds(H + g * L, L)] - t0[g] for g in range(4)]
        gam = [g_v[pl.ds(g * L, L)] for g in range(4)]
        bet = [b_v[pl.ds(g * L, L)] for g in range(4)]

        inv_h = jnp.float32(1.0 / H)

        def per_row(r, carry):
            rg = row0 + r
            base = rg * S
            pltpu.sync_copy(ids_hbm.at[pl.ds(base, c_len)], idx_v.at[0])
            pltpu.sync_copy(ids_hbm.at[pl.ds(base + s2, c_len)], idx_v.at[1])
            pltpu.sync_copy(tt_hbm.at[pl.ds(base, S)], tt_v)
            cp1 = pltpu.async_copy(
                tok_table_hbm.at[idx_v.at[0]], tok_v.at[pl.ds(0, c_len)], sem)
            cp2 = pltpu.async_copy(
                tok_table_hbm.at[idx_v.at[1]], tok_v.at[pl.ds(s2, c_len)], sem)
            cp1.wait()
            cp2.wait()

            def per_tok(s, carry2):
                ttf = lax.broadcast(tt_v[s].astype(jnp.float32), (L,))
                e = []
                for g in range(4):
                    tok_g = tok_v[s, pl.ds(g * L, L)]
                    pos_g = pos_v[pl.ds(s * H + g * L, L)]
                    e.append(tok_g + pos_g + t0[g] + ttf * td[g])
                sum4 = (e[0] + e[1]) + (e[2] + e[3])
                q4 = (e[0] * e[0] + e[1] * e[1]) + (e[2] * e[2] + e[3] * e[3])
                mean = jnp.sum(sum4) * inv_h
                var = jnp.sum(q4) * inv_h - mean * mean
                mv = lax.broadcast(mean, (L,))
                r16 = _rsqrt16(lax.broadcast(var + eps, (L,)))
                for g in range(4):
                    out_v[pl.ds(s * H + g * L, L)] = (
                        (e[g] - mv) * r16 * gam[g] + bet[g])
                return carry2

            lax.fori_loop(0, S, per_tok, 0, unroll=2)
            pltpu.sync_copy(out_v, out_hbm.at[pl.ds(base * H, S * H)])
            return carry

        lax.fori_loop(0, rows_per_w, per_row, 0)

    return k


def kernel(input_ids, token_type_ids, token_table, pos_table, type_table,
           gamma, beta):
    B, S = input_ids.shape
    V, H = token_table.shape
    eps = jnp.float32(1e-5)
    k = _make_kernel(B, S, H, V, eps)
    out_flat = k(
        input_ids.reshape(-1),
        token_type_ids.reshape(-1),
        token_table,
        pos_table[:S].reshape(-1),
        type_table.reshape(-1),
        gamma,
        beta,
    )
    return out_flat.reshape(B, S, H)


# speedup vs baseline: 1.3339x; 1.3339x over previous
import functools
import jax
import jax.numpy as jnp
from jax import lax
from jax.experimental import pallas as pl
from jax.experimental.pallas import tpu as pltpu
from jax.experimental.pallas import tpu_sc as plsc

L = 16

_GATHER_DN = lax.GatherDimensionNumbers(
    offset_dims=(), collapsed_slice_dims=(0,), start_index_map=(0,))


def _lane_sum(v):
    for k2 in (1, 2, 4, 8):
        perm = (jnp.arange(L, dtype=jnp.int32) ^ k2).reshape(L, 1)
        v = v + lax.gather(v, perm, _GATHER_DN, (1,),
                           mode=lax.GatherScatterMode.PROMISE_IN_BOUNDS)
    return v


def _rsqrt16(x):
    i = lax.bitcast_convert_type(x, jnp.int32)
    i = jnp.full((L,), 0x5F3759DF, dtype=jnp.int32) - lax.shift_right_logical(
        i, jnp.full((L,), 1, dtype=jnp.int32))
    y = lax.bitcast_convert_type(i, jnp.float32)
    half = jnp.full((L,), 0.5, dtype=jnp.float32)
    three_half = jnp.full((L,), 1.5, dtype=jnp.float32)
    hx = half * x
    for _ in range(3):
        y = y * (three_half - hx * y * y)
    return y


def _make_kernel(B, S, H, V, eps):
    assert H == 4 * L
    info = plsc.get_sparse_core_info()
    nw = info.num_cores * info.num_subcores
    assert B % nw == 0
    rows_per_w = B // nw
    c_len = ((S + 1) // 2 + 7) // 8 * 8
    s2 = S - c_len
    assert s2 >= 0 and s2 % 8 == 0 and c_len <= 128 and S % 8 == 0

    mesh = plsc.VectorSubcoreMesh(core_axis_name="c", subcore_axis_name="s")

    @functools.partial(
        pl.kernel,
        out_type=jax.ShapeDtypeStruct((B * S * H,), jnp.float32),
        mesh=mesh,
        scratch_types=[
            pltpu.VMEM((2, c_len), jnp.int32),
            pltpu.VMEM((S + L,), jnp.int32),
            pltpu.VMEM((S, H), jnp.float32),
            pltpu.VMEM((S * H,), jnp.float32),
            pltpu.VMEM((2 * H,), jnp.float32),
            pltpu.VMEM((H,), jnp.float32),
            pltpu.VMEM((H,), jnp.float32),
            pltpu.VMEM((S * H,), jnp.float32),
            pltpu.SemaphoreType.DMA,
        ],
        compiler_params=pltpu.CompilerParams(use_tc_tiling_on_sc=False),
    )
    def k(ids_hbm, tt_hbm, tok_table_hbm, pos_hbm, typ_hbm, g_hbm, b_hbm,
          out_hbm, idx_v, tt_v, tok_v, pos_v, typ_v, g_v, b_v, out_v, sem):
        wid = lax.axis_index("s") * info.num_cores + lax.axis_index("c")
        row0 = wid * rows_per_w

        pltpu.sync_copy(pos_hbm, pos_v)
        pltpu.sync_copy(typ_hbm, typ_v)
        pltpu.sync_copy(g_hbm, g_v)
        pltpu.sync_copy(b_hbm, b_v)

        t0 = [typ_v[pl.ds(g * L, L)] for g in range(4)]
        td = [typ_v[pl.ds(H + g * L, L)] - t0[g] for g in range(4)]
        gam = [g_v[pl.ds(g * L, L)] for g in range(4)]
        bet = [b_v[pl.ds(g * L, L)] for g in range(4)]

        inv_h = jnp.float32(1.0 / H)

        def per_row(r, carry):
            rg = row0 + r
            base = rg * S
            pltpu.sync_copy(ids_hbm.at[pl.ds(base, c_len)], idx_v.at[0])
            pltpu.sync_copy(ids_hbm.at[pl.ds(base + s2, c_len)], idx_v.at[1])
            pltpu.sync_copy(tt_hbm.at[pl.ds(base, S)], tt_v.at[pl.ds(0, S)])
            cp1 = pltpu.async_copy(
                tok_table_hbm.at[idx_v.at[0]], tok_v.at[pl.ds(0, c_len)], sem)
            cp2 = pltpu.async_copy(
                tok_table_hbm.at[idx_v.at[1]], tok_v.at[pl.ds(s2, c_len)], sem)
            cp1.wait()
            cp2.wait()

            def per_tok(s, carry2):
                tt16 = tt_v[pl.ds(s, L)]
                ttf = lax.broadcast(tt16[0].astype(jnp.float32), (L,))
                e = []
                for g in range(4):
                    tok_g = tok_v[s, pl.ds(g * L, L)]
                    pos_g = pos_v[pl.ds(s * H + g * L, L)]
                    e.append(tok_g + pos_g + t0[g] + ttf * td[g])
                sum4 = (e[0] + e[1]) + (e[2] + e[3])
                q4 = (e[0] * e[0] + e[1] * e[1]) + (e[2] * e[2] + e[3] * e[3])
                mv = _lane_sum(sum4) * inv_h
                var = _lane_sum(q4) * inv_h - mv * mv
                r16 = _rsqrt16(var + eps)
                for g in range(4):
                    out_v[pl.ds(s * H + g * L, L)] = (
                        (e[g] - mv) * r16 * gam[g] + bet[g])
                return carry2

            lax.fori_loop(0, S, per_tok, 0, unroll=2)
            pltpu.sync_copy(out_v, out_hbm.at[pl.ds(base * H, S * H)])
            return carry

        lax.fori_loop(0, rows_per_w, per_row, 0)

    return k


def kernel(input_ids, token_type_ids, token_table, pos_table, type_table,
           gamma, beta):
    B, S = input_ids.shape
    V, H = token_table.shape
    eps = jnp.float32(1e-5)
    k = _make_kernel(B, S, H, V, eps)
    out_flat = k(
        input_ids.reshape(-1),
        token_type_ids.reshape(-1),
        token_table,
        pos_table[:S].reshape(-1),
        type_table.reshape(-1),
        gamma,
        beta,
    )
    return out_flat.reshape(B, S, H)


# speedup vs baseline: 1.5805x; 1.1849x over previous
import functools
import jax
import jax.numpy as jnp
from jax import lax
from jax.experimental import pallas as pl
from jax.experimental.pallas import tpu as pltpu
from jax.experimental.pallas import tpu_sc as plsc

L = 16

_GATHER_DN = lax.GatherDimensionNumbers(
    offset_dims=(), collapsed_slice_dims=(0,), start_index_map=(0,))


def _lane_sum(v):
    for k2 in (1, 2, 4, 8):
        perm = (jnp.arange(L, dtype=jnp.int32) ^ k2).reshape(L, 1)
        v = v + lax.gather(v, perm, _GATHER_DN, (1,),
                           mode=lax.GatherScatterMode.PROMISE_IN_BOUNDS)
    return v


def _rsqrt16(x):
    i = lax.bitcast_convert_type(x, jnp.int32)
    i = jnp.full((L,), 0x5F3759DF, dtype=jnp.int32) - lax.shift_right_logical(
        i, jnp.full((L,), 1, dtype=jnp.int32))
    y = lax.bitcast_convert_type(i, jnp.float32)
    half = jnp.full((L,), 0.5, dtype=jnp.float32)
    three_half = jnp.full((L,), 1.5, dtype=jnp.float32)
    hx = half * x
    for _ in range(2):
        y = y * (three_half - hx * y * y)
    return y


def _make_kernel(B, S, H, V, eps):
    assert H == 4 * L
    info = plsc.get_sparse_core_info()
    nw = info.num_cores * info.num_subcores
    assert B % nw == 0
    rows_per_w = B // nw
    assert rows_per_w % 2 == 0
    c_len = ((S + 1) // 2 + 7) // 8 * 8
    s2 = S - c_len
    assert s2 >= 0 and s2 % 8 == 0 and c_len <= 128 and S % 8 == 0

    mesh = plsc.VectorSubcoreMesh(core_axis_name="c", subcore_axis_name="s")

    @functools.partial(
        pl.kernel,
        out_type=jax.ShapeDtypeStruct((B * S * H,), jnp.float32),
        mesh=mesh,
        scratch_types=[
            pltpu.VMEM((2, 2, c_len), jnp.int32),
            pltpu.VMEM((2, S + L), jnp.int32),
            pltpu.VMEM((2, S, H), jnp.float32),
            pltpu.VMEM((S * H,), jnp.float32),
            pltpu.VMEM((2 * H,), jnp.float32),
            pltpu.VMEM((H,), jnp.float32),
            pltpu.VMEM((H,), jnp.float32),
            pltpu.VMEM((2, S * H), jnp.float32),
            pltpu.SemaphoreType.DMA,
            pltpu.SemaphoreType.DMA,
            pltpu.SemaphoreType.DMA,
            pltpu.SemaphoreType.DMA,
            pltpu.SemaphoreType.DMA,
            pltpu.SemaphoreType.DMA,
        ],
        compiler_params=pltpu.CompilerParams(use_tc_tiling_on_sc=False),
    )
    def k(ids_hbm, tt_hbm, tok_table_hbm, pos_hbm, typ_hbm, g_hbm, b_hbm,
          out_hbm, idx_v, tt_v, tok_v, pos_v, typ_v, g_v, b_v, out_v,
          sem_idx0, sem_idx1, sem_gat0, sem_gat1, sem_out0, sem_out1):
        sem_idx = (sem_idx0, sem_idx1)
        sem_gat = (sem_gat0, sem_gat1)
        sem_out = (sem_out0, sem_out1)
        wid = lax.axis_index("s") * info.num_cores + lax.axis_index("c")
        row0 = wid * rows_per_w

        pltpu.sync_copy(pos_hbm, pos_v)
        pltpu.sync_copy(typ_hbm, typ_v)
        pltpu.sync_copy(g_hbm, g_v)
        pltpu.sync_copy(b_hbm, b_v)

        t0 = [typ_v[pl.ds(g * L, L)] for g in range(4)]
        td = [typ_v[pl.ds(H + g * L, L)] - t0[g] for g in range(4)]
        gam = [g_v[pl.ds(g * L, L)] for g in range(4)]
        bet = [b_v[pl.ds(g * L, L)] for g in range(4)]

        def fold(s, carry):
            for g in range(4):
                sl = pl.ds(s * H + g * L, L)
                pos_v[sl] = pos_v[sl] + t0[g]
            return carry

        lax.fori_loop(0, S, fold, 0, unroll=4)

        inv_h = jnp.float32(1.0 / H)

        def id_chunk_copies(row, b):
            base = (row0 + row) * S
            return (
                pltpu.make_async_copy(
                    ids_hbm.at[pl.ds(base, c_len)], idx_v.at[b, 0],
                    sem_idx[b]),
                pltpu.make_async_copy(
                    ids_hbm.at[pl.ds(base + s2, c_len)], idx_v.at[b, 1],
                    sem_idx[b]),
            )

        def tt_copy(row, b):
            base = (row0 + row) * S
            return pltpu.make_async_copy(
                tt_hbm.at[pl.ds(base, S)], tt_v.at[b, pl.ds(0, S)],
                sem_idx[b])

        def idx_copies(row, b):
            return id_chunk_copies(row, b) + (tt_copy(row, b),)

        def gat_copies(b):
            return (
                pltpu.make_async_copy(
                    tok_table_hbm.at[idx_v.at[b, 0]],
                    tok_v.at[b, pl.ds(0, c_len)], sem_gat[b]),
                pltpu.make_async_copy(
                    tok_table_hbm.at[idx_v.at[b, 1]],
                    tok_v.at[b, pl.ds(s2, c_len)], sem_gat[b]),
            )

        def out_copy(row, b):
            base = (row0 + row) * S
            return pltpu.make_async_copy(
                out_v.at[b], out_hbm.at[pl.ds(base * H, S * H)],
                sem_out[b])

        def compute(row, b):
            tokb = tok_v.at[b]
            ttb = tt_v.at[b]
            outb = out_v.at[b]

            def per_tok(s, carry2):
                tt16 = ttb[pl.ds(s, L)]
                ttf = lax.broadcast(tt16[0].astype(jnp.float32), (L,))
                e = []
                for g in range(4):
                    tok_g = tokb[s, pl.ds(g * L, L)]
                    pos_g = pos_v[pl.ds(s * H + g * L, L)]
                    e.append(tok_g + pos_g + ttf * td[g])
                sum4 = (e[0] + e[1]) + (e[2] + e[3])
                q4 = (e[0] * e[0] + e[1] * e[1]) + (e[2] * e[2] + e[3] * e[3])
                mv = _lane_sum(sum4) * inv_h
                var = _lane_sum(q4) * inv_h - mv * mv
                r16 = _rsqrt16(var + eps)
                for g in range(4):
                    outb[pl.ds(s * H + g * L, L)] = (
                        (e[g] - mv) * r16 * gam[g] + bet[g])
                return carry2

            lax.fori_loop(0, S, per_tok, 0, unroll=2)

        for c in idx_copies(0, 0):
            c.start()
        for c in idx_copies(1, 1):
            c.start()
        for c in idx_copies(0, 0):
            c.wait()
        for c in gat_copies(0):
            c.start()

        def outer(i, carry):
            r0 = i * 2
            for b in (0, 1):
                row = r0 + b
                other = 1 - b

                @pl.when(row + 1 < rows_per_w)
                def _():
                    for c in idx_copies(row + 1, other):
                        c.wait()
                    for c in gat_copies(other):
                        c.start()

                for c in gat_copies(b):
                    c.wait()

                @pl.when(row + 2 < rows_per_w)
                def _():
                    for c in id_chunk_copies(row + 2, b):
                        c.start()

                @pl.when(row >= 2)
                def _():
                    out_copy(row - 2, b).wait()

                compute(row, b)

                @pl.when(row + 2 < rows_per_w)
                def _():
                    tt_copy(row + 2, b).start()

                out_copy(row, b).start()
            return carry

        lax.fori_loop(0, rows_per_w // 2, outer, 0)
        out_copy(rows_per_w - 2, 0).wait()
        out_copy(rows_per_w - 1, 1).wait()

    return k


def kernel(input_ids, token_type_ids, token_table, pos_table, type_table,
           gamma, beta):
    B, S = input_ids.shape
    V, H = token_table.shape
    eps = jnp.float32(1e-5)
    k = _make_kernel(B, S, H, V, eps)
    out_flat = k(
        input_ids.reshape(-1),
        token_type_ids.reshape(-1),
        token_table,
        pos_table[:S].reshape(-1),
        type_table.reshape(-1),
        gamma,
        beta,
    )
    return out_flat.reshape(B, S, H)


# speedup vs baseline: 2.0713x; 1.3105x over previous
import functools
import jax
import jax.numpy as jnp
from jax import lax
from jax.experimental import pallas as pl
from jax.experimental.pallas import tpu as pltpu
from jax.experimental.pallas import tpu_sc as plsc

L = 16

_GATHER_DN = lax.GatherDimensionNumbers(
    offset_dims=(), collapsed_slice_dims=(0,), start_index_map=(0,))


def _lane_sum(v):
    for k2 in (1, 2, 4, 8):
        perm = (jnp.arange(L, dtype=jnp.int32) ^ k2).reshape(L, 1)
        v = v + lax.gather(v, perm, _GATHER_DN, (1,),
                           mode=lax.GatherScatterMode.PROMISE_IN_BOUNDS)
    return v


def _splat(v, j):
    idx = jnp.full((L, 1), j, dtype=jnp.int32)
    return lax.gather(v, idx, _GATHER_DN, (1,),
                      mode=lax.GatherScatterMode.PROMISE_IN_BOUNDS)


def _rsqrt16(x):
    i = lax.bitcast_convert_type(x, jnp.int32)
    i = jnp.full((L,), 0x5F3759DF, dtype=jnp.int32) - lax.shift_right_logical(
        i, jnp.full((L,), 1, dtype=jnp.int32))
    y = lax.bitcast_convert_type(i, jnp.float32)
    half = jnp.full((L,), 0.5, dtype=jnp.float32)
    three_half = jnp.full((L,), 1.5, dtype=jnp.float32)
    hx = half * x
    for _ in range(2):
        y = y * (three_half - hx * y * y)
    return y


def _make_kernel(B, S, H, V, eps):
    assert H == 4 * L
    info = plsc.get_sparse_core_info()
    nw = info.num_cores * info.num_subcores
    assert B % nw == 0
    rows_per_w = B // nw
    assert rows_per_w % 2 == 0
    c_len = ((S + 1) // 2 + 7) // 8 * 8
    s2 = S - c_len
    assert s2 >= 0 and s2 % 8 == 0 and c_len <= 128 and S % 8 == 0

    mesh = plsc.VectorSubcoreMesh(core_axis_name="c", subcore_axis_name="s")

    @functools.partial(
        pl.kernel,
        out_type=jax.ShapeDtypeStruct((B * S * H,), jnp.float32),
        mesh=mesh,
        scratch_types=[
            pltpu.VMEM((2, 2, c_len), jnp.int32),
            pltpu.VMEM((2, S + L), jnp.int32),
            pltpu.VMEM((2, S, H), jnp.float32),
            pltpu.VMEM((S * H,), jnp.float32),
            pltpu.VMEM((2 * H,), jnp.float32),
            pltpu.VMEM((H,), jnp.float32),
            pltpu.VMEM((H,), jnp.float32),
            pltpu.VMEM((2, S * H), jnp.float32),
            pltpu.VMEM((L * L,), jnp.float32),
            pltpu.VMEM((L * L,), jnp.float32),
            pltpu.SemaphoreType.DMA,
            pltpu.SemaphoreType.DMA,
            pltpu.SemaphoreType.DMA,
            pltpu.SemaphoreType.DMA,
            pltpu.SemaphoreType.DMA,
            pltpu.SemaphoreType.DMA,
        ],
        compiler_params=pltpu.CompilerParams(
            use_tc_tiling_on_sc=False, needs_layout_passes=False),
    )
    def k(ids_hbm, tt_hbm, tok_table_hbm, pos_hbm, typ_hbm, g_hbm, b_hbm,
          out_hbm, idx_v, tt_v, tok_v, pos_v, typ_v, g_v, b_v, out_v,
          ssum_v, sq_v,
          sem_idx0, sem_idx1, sem_gat0, sem_gat1, sem_out0, sem_out1):
        sem_idx = (sem_idx0, sem_idx1)
        sem_gat = (sem_gat0, sem_gat1)
        sem_out = (sem_out0, sem_out1)
        wid = lax.axis_index("s") * info.num_cores + lax.axis_index("c")
        row0 = wid * rows_per_w

        pltpu.sync_copy(pos_hbm, pos_v)
        pltpu.sync_copy(typ_hbm, typ_v)
        pltpu.sync_copy(g_hbm, g_v)
        pltpu.sync_copy(b_hbm, b_v)

        t0 = [typ_v[pl.ds(g * L, L)] for g in range(4)]
        td = [typ_v[pl.ds(H + g * L, L)] - t0[g] for g in range(4)]
        gam = [g_v[pl.ds(g * L, L)] for g in range(4)]
        bet = [b_v[pl.ds(g * L, L)] for g in range(4)]

        def fold(s, carry):
            for g in range(4):
                sl = pl.ds(s * H + g * L, L)
                pos_v[sl] = pos_v[sl] + t0[g]
            return carry

        lax.fori_loop(0, S, fold, 0, unroll=4)

        inv_h = jnp.float32(1.0 / H)

        def id_chunk_copies(row, b):
            base = (row0 + row) * S
            return (
                pltpu.make_async_copy(
                    ids_hbm.at[pl.ds(base, c_len)], idx_v.at[b, 0],
                    sem_idx[b]),
                pltpu.make_async_copy(
                    ids_hbm.at[pl.ds(base + s2, c_len)], idx_v.at[b, 1],
                    sem_idx[b]),
            )

        def tt_copy(row, b):
            base = (row0 + row) * S
            return pltpu.make_async_copy(
                tt_hbm.at[pl.ds(base, S)], tt_v.at[b, pl.ds(0, S)],
                sem_idx[b])

        def idx_copies(row, b):
            return id_chunk_copies(row, b) + (tt_copy(row, b),)

        def gat_copies(b):
            return (
                pltpu.make_async_copy(
                    tok_table_hbm.at[idx_v.at[b, 0]],
                    tok_v.at[b, pl.ds(0, c_len)], sem_gat[b]),
                pltpu.make_async_copy(
                    tok_table_hbm.at[idx_v.at[b, 1]],
                    tok_v.at[b, pl.ds(s2, c_len)], sem_gat[b]),
            )

        def out_copy(row, b):
            base = (row0 + row) * S
            return pltpu.make_async_copy(
                out_v.at[b], out_hbm.at[pl.ds(base * H, S * H)],
                sem_out[b])

        n_chunks = -(-S // L)
        iota16 = lax.iota(jnp.int32, L) * L

        def compute(row, b):
            tokb = tok_v.at[b]
            ttb = tt_v.at[b]
            outb = out_v.at[b]

            def per_chunk(c, carry2):
                s0 = jnp.minimum(c * L, S - L)
                base = s0 * H
                ttf16 = ttb[pl.ds(s0, L)].astype(jnp.float32)
                for j in range(L):
                    ttf = _splat(ttf16, j)
                    e = []
                    for g in range(4):
                        tok_g = tokb[s0 + j, pl.ds(g * L, L)]
                        pos_g = pos_v[pl.ds(base + j * H + g * L, L)]
                        e.append(tok_g + pos_g + ttf * td[g])
                    for g in range(4):
                        outb[pl.ds(base + j * H + g * L, L)] = e[g]
                    sum4 = (e[0] + e[1]) + (e[2] + e[3])
                    q4 = (e[0] * e[0] + e[1] * e[1]) + (
                        e[2] * e[2] + e[3] * e[3])
                    ssum_v[pl.ds(j * L, L)] = sum4
                    sq_v[pl.ds(j * L, L)] = q4
                acc_s = plsc.load_gather(ssum_v, [iota16])
                acc_q = plsc.load_gather(sq_v, [iota16])
                for l in range(1, L):
                    idx = iota16 + l
                    acc_s = acc_s + plsc.load_gather(ssum_v, [idx])
                    acc_q = acc_q + plsc.load_gather(sq_v, [idx])
                mean = acc_s * inv_h
                var = acc_q * inv_h - mean * mean
                rstd = _rsqrt16(var + eps)
                for j in range(L):
                    m_s = _splat(mean, j)
                    r_s = _splat(rstd, j)
                    for g in range(4):
                        sl = pl.ds(base + j * H + g * L, L)
                        outb[sl] = (outb[sl] - m_s) * (r_s * gam[g]) + bet[g]
                return carry2

            lax.fori_loop(0, n_chunks, per_chunk, 0)

        for c in idx_copies(0, 0):
            c.start()
        for c in idx_copies(1, 1):
            c.start()
        for c in idx_copies(0, 0):
            c.wait()
        for c in gat_copies(0):
            c.start()

        def outer(i, carry):
            r0 = i * 2
            for b in (0, 1):
                row = r0 + b
                other = 1 - b

                @pl.when(row + 1 < rows_per_w)
                def _():
                    for c in idx_copies(row + 1, other):
                        c.wait()
                    for c in gat_copies(other):
                        c.start()

                for c in gat_copies(b):
                    c.wait()

                @pl.when(row + 2 < rows_per_w)
                def _():
                    for c in id_chunk_copies(row + 2, b):
                        c.start()

                @pl.when(row >= 2)
                def _():
                    out_copy(row - 2, b).wait()

                compute(row, b)

                @pl.when(row + 2 < rows_per_w)
                def _():
                    tt_copy(row + 2, b).start()

                out_copy(row, b).start()
            return carry

        lax.fori_loop(0, rows_per_w // 2, outer, 0)
        out_copy(rows_per_w - 2, 0).wait()
        out_copy(rows_per_w - 1, 1).wait()

    return k


def kernel(input_ids, token_type_ids, token_table, pos_table, type_table,
           gamma, beta):
    B, S = input_ids.shape
    V, H = token_table.shape
    eps = jnp.float32(1e-5)
    k = _make_kernel(B, S, H, V, eps)
    out_flat = k(
        input_ids.reshape(-1),
        token_type_ids.reshape(-1),
        token_table,
        pos_table[:S].reshape(-1),
        type_table.reshape(-1),
        gamma,
        beta,
    )
    return out_flat.reshape(B, S, H)
